# Initial kernel scaffold; baseline (speedup 1.0000x reference)
import jax
import jax.numpy as jnp
from jax.experimental import pallas as pl


def kernel(x, edge_index, edge_weight, bn_gamma, bn_beta, W1, b1, W2, b2, fc1_W, fc1_b, fc2_W, fc2_b):
    raise NotImplementedError("write your pallas kernel here")


# speedup vs baseline: 39.3988x; 39.3988x over previous
import functools
import jax
import jax.numpy as jnp
from jax.experimental import pallas as pl
from jax.experimental.pallas import tpu as pltpu


def _celu(v):
    return jnp.where(v > 0, v, jnp.exp(jnp.minimum(v, 0.0)) - 1.0)


def _stats_kernel(x_ref, gamma_ref, beta_ref, scale_ref, shift_ref, acc_ref, *, nsteps, n_rows):
    i = pl.program_id(0)

    @pl.when(i == 0)
    def _init():
        acc_ref[...] = jnp.zeros_like(acc_ref)

    xb = x_ref[...]
    acc_ref[0:1, :] += jnp.sum(xb, axis=0, keepdims=True)
    acc_ref[1:2, :] += jnp.sum(xb * xb, axis=0, keepdims=True)

    @pl.when(i == nsteps - 1)
    def _fin():
        inv_n = 1.0 / n_rows
        mean = acc_ref[0:1, :] * inv_n
        var = acc_ref[1:2, :] * inv_n - mean * mean
        rstd = jax.lax.rsqrt(var + 1e-5)
        sc = gamma_ref[...] * rstd
        scale_ref[...] = sc
        shift_ref[...] = beta_ref[...] - mean * sc


def _gcn_kernel(x_ref, src_ref, dst_ref, ew_ref, scale_ref, shift_ref,
                w1_ref, b1_ref, w2_ref, b2_ref, out_ref, *, bt, n_per, e_per):
    f32 = jnp.float32
    xb = x_ref[...] * scale_ref[...] + shift_ref[...]
    h1 = jnp.dot(xb, w1_ref[...], preferred_element_type=f32)
    ew = ew_ref[...]
    iota_n = jax.lax.broadcasted_iota(jnp.int32, (n_per, e_per), 0)
    r_idx = jax.lax.broadcasted_iota(jnp.int32, (n_per, n_per), 0)
    c_idx = jax.lax.broadcasted_iota(jnp.int32, (n_per, n_per), 1)
    eye = r_idx == c_idx
    for b in range(bt):
        src = src_ref[0, :, b * e_per:(b + 1) * e_per] & (n_per - 1)
        dst = dst_ref[0, :, b * e_per:(b + 1) * e_per] & (n_per - 1)
        dt = (iota_n == dst).astype(f32)
        st = (iota_n == src).astype(f32)
        deg = jnp.sum(dt * ew, axis=1, keepdims=True) + 1.0
        dinv = jnp.where(deg > 0, jax.lax.rsqrt(deg), 0.0)
        dsrc = jnp.sum(st * dinv, axis=0, keepdims=True)
        ddst = jnp.sum(dt * dinv, axis=0, keepdims=True)
        norm = dsrc * ew * ddst
        m = st * norm
        a_hat = jax.lax.dot_general(dt, m, (((1,), (1,)), ((), ())),
                                    preferred_element_type=f32)
        a_hat = a_hat + jnp.where(eye, dinv * dinv, 0.0)
        h1b = h1[b * n_per:(b + 1) * n_per, :]
        c1 = _celu(jnp.dot(a_hat, h1b, preferred_element_type=f32) + b1_ref[...])
        h2b = jnp.dot(c1, w2_ref[...], preferred_element_type=f32)
        c2 = _celu(jnp.dot(a_hat, h2b, preferred_element_type=f32) + b2_ref[...])
        out_ref[b * n_per:(b + 1) * n_per, :] = c2


def _fc_kernel(g_ref, w1_ref, b1_ref, w2_ref, b2_ref, out_ref):
    f32 = jnp.float32
    h = _celu(jnp.dot(g_ref[...], w1_ref[...], preferred_element_type=f32) + b1_ref[...])
    out_ref[...] = jnp.dot(h, w2_ref[...], preferred_element_type=f32) + b2_ref[...]


@jax.jit
def kernel(x, edge_index, edge_weight, bn_gamma, bn_beta, W1, b1, W2, b2, fc1_W, fc1_b, fc2_W, fc2_b):
    f32 = jnp.float32
    n, f = x.shape
    g1 = W1.shape[1]
    g2 = W2.shape[1]
    n_per = fc1_W.shape[0] // g2
    b_graphs = n // n_per
    e_per = edge_weight.shape[0]

    bt = 16 if b_graphs % 16 == 0 else 1
    nsteps = b_graphs // bt
    rows = bt * n_per

    gamma2 = bn_gamma.reshape(1, f)
    beta2 = bn_beta.reshape(1, f)
    scale, shift = pl.pallas_call(
        functools.partial(_stats_kernel, nsteps=nsteps, n_rows=float(n)),
        grid=(nsteps,),
        in_specs=[
            pl.BlockSpec((rows, f), lambda i: (i, 0)),
            pl.BlockSpec((1, f), lambda i: (0, 0)),
            pl.BlockSpec((1, f), lambda i: (0, 0)),
        ],
        out_specs=[
            pl.BlockSpec((1, f), lambda i: (0, 0)),
            pl.BlockSpec((1, f), lambda i: (0, 0)),
        ],
        out_shape=[
            jax.ShapeDtypeStruct((1, f), f32),
            jax.ShapeDtypeStruct((1, f), f32),
        ],
        scratch_shapes=[pltpu.VMEM((2, f), f32)],
    )(x, gamma2, beta2)

    src3 = edge_index[0].reshape(nsteps, 1, bt * e_per)
    dst3 = edge_index[1].reshape(nsteps, 1, bt * e_per)
    ew2 = edge_weight.reshape(1, e_per)

    c2 = pl.pallas_call(
        functools.partial(_gcn_kernel, bt=bt, n_per=n_per, e_per=e_per),
        grid=(nsteps,),
        in_specs=[
            pl.BlockSpec((rows, f), lambda i: (i, 0)),
            pl.BlockSpec((1, 1, bt * e_per), lambda i: (i, 0, 0)),
            pl.BlockSpec((1, 1, bt * e_per), lambda i: (i, 0, 0)),
            pl.BlockSpec((1, e_per), lambda i: (0, 0)),
            pl.BlockSpec((1, f), lambda i: (0, 0)),
            pl.BlockSpec((1, f), lambda i: (0, 0)),
            pl.BlockSpec((f, g1), lambda i: (0, 0)),
            pl.BlockSpec((1, g1), lambda i: (0, 0)),
            pl.BlockSpec((g1, g2), lambda i: (0, 0)),
            pl.BlockSpec((1, g2), lambda i: (0, 0)),
        ],
        out_specs=pl.BlockSpec((rows, g2), lambda i: (i, 0)),
        out_shape=jax.ShapeDtypeStruct((n, g2), f32),
    )(x, src3, dst3, ew2, scale, shift, W1, b1.reshape(1, g1), W2, b2.reshape(1, g2))

    g = c2.reshape(b_graphs, n_per * g2)

    fc1_n = fc1_W.shape[1]
    out_n = fc2_W.shape[1]
    fc_bt = 256 if b_graphs % 256 == 0 else b_graphs
    logits = pl.pallas_call(
        _fc_kernel,
        grid=(b_graphs // fc_bt,),
        in_specs=[
            pl.BlockSpec((fc_bt, n_per * g2), lambda i: (i, 0)),
            pl.BlockSpec((n_per * g2, fc1_n), lambda i: (0, 0)),
            pl.BlockSpec((1, fc1_n), lambda i: (0, 0)),
            pl.BlockSpec((fc1_n, out_n), lambda i: (0, 0)),
            pl.BlockSpec((1, out_n), lambda i: (0, 0)),
        ],
        out_specs=pl.BlockSpec((fc_bt, out_n), lambda i: (i, 0)),
        out_shape=jax.ShapeDtypeStruct((b_graphs, out_n), f32),
    )(g, fc1_W, fc1_b.reshape(1, fc1_n), fc2_W, fc2_b.reshape(1, out_n))
    return logits
